# Initial kernel scaffold; baseline (speedup 1.0000x reference)
import jax
import jax.numpy as jnp
from jax.experimental import pallas as pl


def kernel(preds, targs):
    raise NotImplementedError("write your pallas kernel here")


# speedup vs baseline: 54.8531x; 54.8531x over previous
import functools

import jax
import jax.numpy as jnp
import numpy as np
from jax import lax
from jax.experimental import pallas as pl
from jax.experimental.pallas import tpu as pltpu

_N = 5000
_NP = 5120
_NB = 40
_BLK = 128
_CUT2 = 4.0
_MATCH2 = 1.0
_BIG = 1e10
_SPLIT = (0.0, 4.0, 8.0 + 1e-05)
_DEG = np.pi / 180.0


def _acos(x):
    ax = jnp.abs(x)
    p = jnp.float32(-0.0012624911)
    for c in (0.0066700901, -0.0170881256, 0.0308918810, -0.0501743046,
              0.0889789874, -0.2145988016, 1.5707963050):
        p = p * ax + jnp.float32(c)
    pos = jnp.sqrt(jnp.maximum(1.0 - ax, 0.0)) * p
    return jnp.where(x >= 0.0, pos, jnp.float32(np.pi) - pos)


def _chan_col(tile, c):
    sel = (lax.broadcasted_iota(jnp.int32, (1, 16), 1) == c).astype(jnp.float32)
    return jnp.sum(tile * sel, axis=1, keepdims=True)


def _analyse_kernel(pd_rows, pd_cols, tg_cols, out, keep_row, tgbest, tgd2,
                    closebb_ref):
    f32 = jnp.float32
    prow_x = pd_rows[0, 0:1, :]
    prow_y = pd_rows[0, 1:2, :]
    prow_z = pd_rows[0, 2:3, :]
    prow_m = pd_rows[0, 3:4, :]
    lane_np = lax.broadcasted_iota(jnp.int32, (1, _NP), 1)
    lane_b = lax.broadcasted_iota(jnp.int32, (1, _BLK), 1)
    sub_b = lax.broadcasted_iota(jnp.int32, (_BLK, 1), 0)
    ident = (lane_b == sub_b).astype(f32)

    keep_row[...] = jnp.zeros((1, _NP), f32)
    alive = jnp.sum(prow_m).astype(jnp.int32)
    nblk = (alive + _BLK - 1) // _BLK

    for k in range(_NB):
        s0 = k * _BLK

        @pl.when(k < nblk)
        def _(k=k, s0=s0):
            ct = pd_cols[0, pl.ds(s0, _BLK), :]
            icx = _chan_col(ct, 0)
            icy = _chan_col(ct, 1)
            icz = _chan_col(ct, 2)
            dx = icx - prow_x
            dy = icy - prow_y
            dz = icz - prow_z
            d2 = dx * dx + dy * dy + dz * dz
            close = (d2 < _CUT2).astype(f32)
            extc = jnp.sum(close * keep_row[...], axis=1, keepdims=True)
            ext01 = (extc > 0.0).astype(f32)
            ext_row = lax.dot_general(
                ext01, ident, (((0,), (0,)), ((), ())),
                preferred_element_type=f32)
            jrx = prow_x[0:1, s0:s0 + _BLK]
            jry = prow_y[0:1, s0:s0 + _BLK]
            jrz = prow_z[0:1, s0:s0 + _BLK]
            bx = icx - jrx
            by = icy - jry
            bz = icz - jrz
            bb = bx * bx + by * by + bz * bz
            closebb_ref[...] = (bb < _CUT2).astype(f32)
            kb0 = prow_m[0:1, s0:s0 + _BLK] * (1.0 - ext_row)

            def step(i, kb):
                row = closebb_ref[pl.ds(i, 1), :]
                lt = (lane_b < i).astype(f32)
                sup = (jnp.sum(row * kb * lt) > 0.0).astype(f32)
                return kb * jnp.where(lane_b == i, 1.0 - sup, 1.0)

            kb = lax.fori_loop(0, _BLK, step, kb0)
            keep_row[0:1, s0:s0 + _BLK] = kb

    keep = keep_row[...]

    def t_tile(tt, carry):
        curmin, curarg = carry
        s = pl.ds(tt * _BLK, _BLK)
        ct = tg_cols[0, s, :]
        tcx = _chan_col(ct, 0)
        tcy = _chan_col(ct, 1)
        tcz = _chan_col(ct, 2)
        tcm = _chan_col(ct, 3)
        dx = tcx - prow_x
        dy = tcy - prow_y
        dz = tcz - prow_z
        d2 = dx * dx + dy * dy + dz * dz
        valid = tcm * keep
        d2m = jnp.where(valid > 0.0, d2, _BIG)
        tmin = jnp.min(d2m, axis=0, keepdims=True)
        tloc = jnp.min(jnp.where(d2m == tmin, sub_b.astype(f32), 1e9),
                       axis=0, keepdims=True)
        tglob = tloc + jnp.float32(1.0) * (tt * _BLK)
        upd = tmin < curmin
        curarg = jnp.where(upd, tglob, curarg)
        curmin = jnp.where(upd, tmin, curmin)
        rmin = jnp.min(d2m, axis=1, keepdims=True)
        ploc = jnp.min(jnp.where(d2m == rmin, lane_np.astype(f32), 1e9),
                       axis=1, keepdims=True)
        tgbest[s, :] = ploc
        tgd2[s, :] = rmin
        return curmin, curarg

    init = (jnp.full((1, _NP), _BIG, f32), jnp.zeros((1, _NP), f32))
    _, pdbest = lax.fori_loop(0, _NB, t_tile, init)

    pdr_rows = [pd_rows[0, 4 + c:5 + c, :] for c in range(6)]
    pdn_rows = [pd_rows[0, 10:11, :], pd_rows[0, 11:12, :]]

    def t3(tt, carry):
        angsum, wcnt, m0, m1, c0, c1 = carry
        s = pl.ds(tt * _BLK, _BLK)
        tgb = tgbest[s, :]
        td2 = tgd2[s, :]
        ct = tg_cols[0, s, :]
        tcz = _chan_col(ct, 2)
        tcm = _chan_col(ct, 3)
        sel = (lane_np.astype(f32) == tgb).astype(f32)
        tgi = sub_b.astype(f32) + jnp.float32(1.0) * (tt * _BLK)
        mut = jnp.sum(sel * (pdbest == tgi).astype(f32), axis=1, keepdims=True)
        w = ((mut > 0.0) & (td2 < _MATCH2)).astype(f32)

        angs = jnp.zeros((_BLK, 1), f32)
        for i in range(2):
            tr = [_chan_col(ct, 4 + 3 * i + c) for c in range(3)]
            dot = (tr[0] * pdr_rows[3 * i + 0] + tr[1] * pdr_rows[3 * i + 1]
                   + tr[2] * pdr_rows[3 * i + 2])
            a_i = jnp.sum(sel * dot, axis=1, keepdims=True)
            n_i = jnp.sum(sel * pdn_rows[i], axis=1, keepdims=True)
            tn_i = jnp.sqrt(tr[0] * tr[0] + tr[1] * tr[1] + tr[2] * tr[2])
            cos = a_i / (n_i * tn_i)
            ang = _acos(jnp.clip(cos, -1.0, 1.0)) / _DEG
            ang = jnp.where(ang != ang, jnp.float32(90.0), ang)
            angs = angs + ang
        angsum = angsum + jnp.sum(w * angs)
        wcnt = wcnt + jnp.sum(w)
        zin0 = ((tcz >= _SPLIT[0]) & (tcz < _SPLIT[1])).astype(f32)
        zin1 = ((tcz >= _SPLIT[1]) & (tcz < _SPLIT[2])).astype(f32)
        m0 = m0 + jnp.sum(w * zin0)
        m1 = m1 + jnp.sum(w * zin1)
        c0 = c0 + jnp.sum(tcm * zin0)
        c1 = c1 + jnp.sum(tcm * zin1)
        return angsum, wcnt, m0, m1, c0, c1

    z6 = (jnp.float32(0.0),) * 6
    angsum, wcnt, m0, m1, tgc0, tgc1 = lax.fori_loop(0, _NB, t3, z6)

    pz0 = ((prow_z >= _SPLIT[0]) & (prow_z < _SPLIT[1])).astype(f32)
    pz1 = ((prow_z >= _SPLIT[1]) & (prow_z < _SPLIT[2])).astype(f32)
    pdc0 = jnp.sum(keep * pz0)
    pdc1 = jnp.sum(keep * pz1)

    wsum = wcnt * 2.0
    ang_mean = jnp.where(wsum > 0.0, angsum / jnp.maximum(wsum, 1.0),
                         jnp.float32(90.0))

    r = lax.broadcasted_iota(jnp.int32, (8, 128), 0)
    c = lax.broadcasted_iota(jnp.int32, (8, 128), 1)
    vals = ((0, 0, m0), (0, 1, pdc0 - m0), (0, 2, tgc0 - m0), (0, 3, ang_mean),
            (1, 0, m1), (1, 1, pdc1 - m1), (1, 2, tgc1 - m1), (1, 3, ang_mean))
    res = jnp.zeros((8, 128), f32)
    for (ri, ci, v) in vals:
        res = jnp.where((r == ri) & (c == ci), v, res)
    out[0] = res


@functools.partial(jax.jit, static_argnames=())
def kernel(preds, targs):
    B = preds.shape[0]
    f32 = jnp.float32
    pred = preds.reshape(B, _N, 10)
    targ = targs.reshape(B, _N, 10)

    gx, gy, gz = jnp.meshgrid(jnp.arange(25), jnp.arange(25), jnp.arange(8),
                              indexing='ij')
    grid = jnp.stack([gx, gy, gz], axis=-1).reshape(-1, 3).astype(f32)
    cell = jnp.asarray([1.0, 1.0, 0.5], dtype=f32)

    def fields(box):
        conf = box[..., 0]
        off = jax.nn.sigmoid(box[..., 1:4])
        pos = (grid + off) * cell
        R = box[..., 4:10]
        return conf, pos, R

    conf_p, pos_p, R_p = fields(pred)
    conf_t, pos_t, R_t = fields(targ)
    mask_p = (conf_p > 0.0).astype(f32)
    mask_t = (conf_t > 0.5).astype(f32)

    order = jnp.argsort(-conf_p, axis=1)
    pos_p = jnp.take_along_axis(pos_p, order[..., None], axis=1)
    R_p = jnp.take_along_axis(R_p, order[..., None], axis=1)
    mask_p = jnp.take_along_axis(mask_p, order, axis=1)
    nrm_p = jnp.linalg.norm(R_p.reshape(B, _N, 2, 3), axis=-1)

    pad = ((0, 0), (0, _NP - _N), (0, 0))

    pd_chan = jnp.concatenate(
        [pos_p, mask_p[..., None], R_p, nrm_p,
         jnp.zeros((B, _N, 4), f32)], axis=-1)
    pd_chan = jnp.pad(pd_chan, pad)
    pd_rows = jnp.swapaxes(pd_chan, 1, 2)
    pd_cols = pd_chan

    tg_chan = jnp.concatenate(
        [pos_t, mask_t[..., None], R_t,
         jnp.zeros((B, _N, 6), f32)], axis=-1)
    tg_cols = jnp.pad(tg_chan, pad)

    out = pl.pallas_call(
        _analyse_kernel,
        grid=(B,),
        in_specs=[
            pl.BlockSpec((1, 16, _NP), lambda b: (b, 0, 0)),
            pl.BlockSpec((1, _NP, 16), lambda b: (b, 0, 0)),
            pl.BlockSpec((1, _NP, 16), lambda b: (b, 0, 0)),
        ],
        out_specs=pl.BlockSpec((1, 8, 128), lambda b: (b, 0, 0)),
        out_shape=jax.ShapeDtypeStruct((B, 8, 128), f32),
        scratch_shapes=[
            pltpu.VMEM((1, _NP), f32),
            pltpu.VMEM((_NP, 1), f32),
            pltpu.VMEM((_NP, 1), f32),
            pltpu.VMEM((_BLK, _BLK), f32),
        ],
    )(pd_rows, pd_cols, tg_cols)

    return out[:, None, :2, :4]


# speedup vs baseline: 106.1645x; 1.9354x over previous
import functools

import jax
import jax.numpy as jnp
import numpy as np
from jax import lax
from jax.experimental import pallas as pl
from jax.experimental.pallas import tpu as pltpu

_N = 5000
_NP = 5120
_NB = 40
_BLK = 128
_CUT2 = 4.0
_MATCH2 = 1.0
_BIG = 1e10
_SPLIT = (0.0, 4.0, 8.0 + 1e-05)
_DEG = np.pi / 180.0


def _acos(x):
    ax = jnp.abs(x)
    p = jnp.float32(-0.0012624911)
    for c in (0.0066700901, -0.0170881256, 0.0308918810, -0.0501743046,
              0.0889789874, -0.2145988016, 1.5707963050):
        p = p * ax + jnp.float32(c)
    pos = jnp.sqrt(jnp.maximum(1.0 - ax, 0.0)) * p
    return jnp.where(x >= 0.0, pos, jnp.float32(np.pi) - pos)


def _chan_col(tile, c):
    sel = (lax.broadcasted_iota(jnp.int32, (1, 16), 1) == c).astype(jnp.float32)
    return jnp.sum(tile * sel, axis=1, keepdims=True)


def _analyse_kernel(pd_rows, pd_cols, tg_cols, out, keep_row, tgbest, tgd2):
    f32 = jnp.float32
    prow_x = pd_rows[0, 0:1, :]
    prow_y = pd_rows[0, 1:2, :]
    prow_z = pd_rows[0, 2:3, :]
    prow_m = pd_rows[0, 3:4, :]
    lane_np = lax.broadcasted_iota(jnp.int32, (1, _NP), 1)
    lane_b = lax.broadcasted_iota(jnp.int32, (1, _BLK), 1)
    sub_b = lax.broadcasted_iota(jnp.int32, (_BLK, 1), 0)
    ident = (lane_b == sub_b).astype(f32)

    keep_row[...] = jnp.zeros((1, _NP), f32)
    alive = jnp.sum(prow_m).astype(jnp.int32)
    nblk = (alive + _BLK - 1) // _BLK

    for k in range(_NB):
        s0 = k * _BLK

        @pl.when(k < nblk)
        def _(k=k, s0=s0):
            ct = pd_cols[0, pl.ds(s0, _BLK), :]
            icx = _chan_col(ct, 0)
            icy = _chan_col(ct, 1)
            icz = _chan_col(ct, 2)
            dx = icx - prow_x
            dy = icy - prow_y
            dz = icz - prow_z
            d2 = dx * dx + dy * dy + dz * dz
            close = (d2 < _CUT2).astype(f32)
            extc = jnp.sum(close * keep_row[...], axis=1, keepdims=True)
            ext01 = (extc > 0.0).astype(f32)
            ext_row = lax.dot_general(
                ext01, ident, (((0,), (0,)), ((), ())),
                preferred_element_type=f32)
            jrx = prow_x[0:1, s0:s0 + _BLK]
            jry = prow_y[0:1, s0:s0 + _BLK]
            jrz = prow_z[0:1, s0:s0 + _BLK]
            bx = icx - jrx
            by = icy - jry
            bz = icz - jrz
            bb = bx * bx + by * by + bz * bz
            trilc = ((bb < _CUT2) & (lane_b < sub_b)).astype(f32)
            kb0 = prow_m[0:1, s0:s0 + _BLK] * (1.0 - ext_row)

            def w_cond(st):
                return st[1]

            def w_body(st):
                kb, _ = st
                sup = lax.dot_general(kb, trilc, (((1,), (1,)), ((), ())),
                                      preferred_element_type=f32)
                kbn = kb0 * (sup == 0.0).astype(f32)
                chg = jnp.sum(jnp.abs(kbn - kb)) > 0.0
                return kbn, chg

            kb, _ = lax.while_loop(w_cond, w_body, (kb0, True))
            keep_row[0:1, s0:s0 + _BLK] = kb

    keep = keep_row[...]

    def t_tile(tt, carry):
        curmin, curarg = carry
        s = pl.ds(tt * _BLK, _BLK)
        ct = tg_cols[0, s, :]
        tcx = _chan_col(ct, 0)
        tcy = _chan_col(ct, 1)
        tcz = _chan_col(ct, 2)
        tcm = _chan_col(ct, 3)
        dx = tcx - prow_x
        dy = tcy - prow_y
        dz = tcz - prow_z
        d2 = dx * dx + dy * dy + dz * dz
        valid = tcm * keep
        d2m = jnp.where(valid > 0.0, d2, _BIG)
        tmin = jnp.min(d2m, axis=0, keepdims=True)
        tloc = jnp.min(jnp.where(d2m == tmin, sub_b.astype(f32), 1e9),
                       axis=0, keepdims=True)
        tglob = tloc + jnp.float32(1.0) * (tt * _BLK)
        upd = tmin < curmin
        curarg = jnp.where(upd, tglob, curarg)
        curmin = jnp.where(upd, tmin, curmin)
        rmin = jnp.min(d2m, axis=1, keepdims=True)
        ploc = jnp.min(jnp.where(d2m == rmin, lane_np.astype(f32), 1e9),
                       axis=1, keepdims=True)
        tgbest[s, :] = ploc
        tgd2[s, :] = rmin
        return curmin, curarg

    init = (jnp.full((1, _NP), _BIG, f32), jnp.zeros((1, _NP), f32))
    _, pdbest = lax.fori_loop(0, _NB, t_tile, init)

    pdr_rows = [pd_rows[0, 4 + c:5 + c, :] for c in range(6)]
    pdn_rows = [pd_rows[0, 10:11, :], pd_rows[0, 11:12, :]]

    def t3(tt, carry):
        angsum, wcnt, m0, m1, c0, c1 = carry
        s = pl.ds(tt * _BLK, _BLK)
        tgb = tgbest[s, :]
        td2 = tgd2[s, :]
        ct = tg_cols[0, s, :]
        tcz = _chan_col(ct, 2)
        tcm = _chan_col(ct, 3)
        sel = (lane_np.astype(f32) == tgb).astype(f32)
        tgi = sub_b.astype(f32) + jnp.float32(1.0) * (tt * _BLK)
        mut = jnp.sum(sel * (pdbest == tgi).astype(f32), axis=1, keepdims=True)
        w = ((mut > 0.0) & (td2 < _MATCH2)).astype(f32)

        angs = jnp.zeros((_BLK, 1), f32)
        for i in range(2):
            tr = [_chan_col(ct, 4 + 3 * i + c) for c in range(3)]
            dot = (tr[0] * pdr_rows[3 * i + 0] + tr[1] * pdr_rows[3 * i + 1]
                   + tr[2] * pdr_rows[3 * i + 2])
            a_i = jnp.sum(sel * dot, axis=1, keepdims=True)
            n_i = jnp.sum(sel * pdn_rows[i], axis=1, keepdims=True)
            tn_i = jnp.sqrt(tr[0] * tr[0] + tr[1] * tr[1] + tr[2] * tr[2])
            cos = a_i / (n_i * tn_i)
            ang = _acos(jnp.clip(cos, -1.0, 1.0)) / _DEG
            ang = jnp.where(ang != ang, jnp.float32(90.0), ang)
            angs = angs + ang
        angsum = angsum + jnp.sum(w * angs)
        wcnt = wcnt + jnp.sum(w)
        zin0 = ((tcz >= _SPLIT[0]) & (tcz < _SPLIT[1])).astype(f32)
        zin1 = ((tcz >= _SPLIT[1]) & (tcz < _SPLIT[2])).astype(f32)
        m0 = m0 + jnp.sum(w * zin0)
        m1 = m1 + jnp.sum(w * zin1)
        c0 = c0 + jnp.sum(tcm * zin0)
        c1 = c1 + jnp.sum(tcm * zin1)
        return angsum, wcnt, m0, m1, c0, c1

    z6 = (jnp.float32(0.0),) * 6
    angsum, wcnt, m0, m1, tgc0, tgc1 = lax.fori_loop(0, _NB, t3, z6)

    pz0 = ((prow_z >= _SPLIT[0]) & (prow_z < _SPLIT[1])).astype(f32)
    pz1 = ((prow_z >= _SPLIT[1]) & (prow_z < _SPLIT[2])).astype(f32)
    pdc0 = jnp.sum(keep * pz0)
    pdc1 = jnp.sum(keep * pz1)

    wsum = wcnt * 2.0
    ang_mean = jnp.where(wsum > 0.0, angsum / jnp.maximum(wsum, 1.0),
                         jnp.float32(90.0))

    r = lax.broadcasted_iota(jnp.int32, (8, 128), 0)
    c = lax.broadcasted_iota(jnp.int32, (8, 128), 1)
    vals = ((0, 0, m0), (0, 1, pdc0 - m0), (0, 2, tgc0 - m0), (0, 3, ang_mean),
            (1, 0, m1), (1, 1, pdc1 - m1), (1, 2, tgc1 - m1), (1, 3, ang_mean))
    res = jnp.zeros((8, 128), f32)
    for (ri, ci, v) in vals:
        res = jnp.where((r == ri) & (c == ci), v, res)
    out[0] = res


@functools.partial(jax.jit, static_argnames=())
def kernel(preds, targs):
    B = preds.shape[0]
    f32 = jnp.float32
    pred = preds.reshape(B, _N, 10)
    targ = targs.reshape(B, _N, 10)

    gx, gy, gz = jnp.meshgrid(jnp.arange(25), jnp.arange(25), jnp.arange(8),
                              indexing='ij')
    grid = jnp.stack([gx, gy, gz], axis=-1).reshape(-1, 3).astype(f32)
    cell = jnp.asarray([1.0, 1.0, 0.5], dtype=f32)

    def fields(box):
        conf = box[..., 0]
        off = jax.nn.sigmoid(box[..., 1:4])
        pos = (grid + off) * cell
        R = box[..., 4:10]
        return conf, pos, R

    conf_p, pos_p, R_p = fields(pred)
    conf_t, pos_t, R_t = fields(targ)
    mask_p = (conf_p > 0.0).astype(f32)
    mask_t = (conf_t > 0.5).astype(f32)

    order = jnp.argsort(-conf_p, axis=1)
    pos_p = jnp.take_along_axis(pos_p, order[..., None], axis=1)
    R_p = jnp.take_along_axis(R_p, order[..., None], axis=1)
    mask_p = jnp.take_along_axis(mask_p, order, axis=1)
    nrm_p = jnp.linalg.norm(R_p.reshape(B, _N, 2, 3), axis=-1)

    pad = ((0, 0), (0, _NP - _N), (0, 0))

    pd_chan = jnp.concatenate(
        [pos_p, mask_p[..., None], R_p, nrm_p,
         jnp.zeros((B, _N, 4), f32)], axis=-1)
    pd_chan = jnp.pad(pd_chan, pad)
    pd_rows = jnp.swapaxes(pd_chan, 1, 2)
    pd_cols = pd_chan

    tg_chan = jnp.concatenate(
        [pos_t, mask_t[..., None], R_t,
         jnp.zeros((B, _N, 6), f32)], axis=-1)
    tg_cols = jnp.pad(tg_chan, pad)

    out = pl.pallas_call(
        _analyse_kernel,
        grid=(B,),
        in_specs=[
            pl.BlockSpec((1, 16, _NP), lambda b: (b, 0, 0)),
            pl.BlockSpec((1, _NP, 16), lambda b: (b, 0, 0)),
            pl.BlockSpec((1, _NP, 16), lambda b: (b, 0, 0)),
        ],
        out_specs=pl.BlockSpec((1, 8, 128), lambda b: (b, 0, 0)),
        out_shape=jax.ShapeDtypeStruct((B, 8, 128), f32),
        scratch_shapes=[
            pltpu.VMEM((1, _NP), f32),
            pltpu.VMEM((_NP, 1), f32),
            pltpu.VMEM((_NP, 1), f32),
        ],
    )(pd_rows, pd_cols, tg_cols)

    return out[:, None, :2, :4]


# speedup vs baseline: 129.3501x; 1.2184x over previous
import functools

import jax
import jax.numpy as jnp
import numpy as np
from jax import lax
from jax.experimental import pallas as pl
from jax.experimental.pallas import tpu as pltpu

_N = 5000
_NP = 5120
_NB = 40
_BLK = 128
_CUT2 = 4.0
_MATCH2 = 1.0
_BIG = 1e10
_SPLIT = (0.0, 4.0, 8.0 + 1e-05)
_DEG = np.pi / 180.0


def _acos(x):
    ax = jnp.abs(x)
    p = jnp.float32(-0.0012624911)
    for c in (0.0066700901, -0.0170881256, 0.0308918810, -0.0501743046,
              0.0889789874, -0.2145988016, 1.5707963050):
        p = p * ax + jnp.float32(c)
    pos = jnp.sqrt(jnp.maximum(1.0 - ax, 0.0)) * p
    return jnp.where(x >= 0.0, pos, jnp.float32(np.pi) - pos)


def _chan_col(tile, c):
    sel = (lax.broadcasted_iota(jnp.int32, (1, 16), 1) == c).astype(jnp.float32)
    return jnp.sum(tile * sel, axis=1, keepdims=True)


def _analyse_kernel(pd_rows, pd_cols, tg_cols, out, keep_row, tgbest, tgd2):
    f32 = jnp.float32
    prow_x = pd_rows[0, 0:1, :]
    prow_y = pd_rows[0, 1:2, :]
    prow_z = pd_rows[0, 2:3, :]
    prow_m = pd_rows[0, 3:4, :]
    lane_np = lax.broadcasted_iota(jnp.int32, (1, _NP), 1)
    lane_b = lax.broadcasted_iota(jnp.int32, (1, _BLK), 1)
    sub_b = lax.broadcasted_iota(jnp.int32, (_BLK, 1), 0)
    ident = (lane_b == sub_b).astype(f32)

    keep_row[...] = jnp.zeros((1, _NP), f32)
    alive = jnp.sum(prow_m).astype(jnp.int32)
    nblk = (alive + _BLK - 1) // _BLK

    for k in range(_NB):
        s0 = k * _BLK

        @pl.when(k < nblk)
        def _(k=k, s0=s0):
            ct = pd_cols[0, pl.ds(s0, _BLK), :]
            icx = _chan_col(ct, 0)
            icy = _chan_col(ct, 1)
            icz = _chan_col(ct, 2)
            dx = icx - prow_x
            dy = icy - prow_y
            dz = icz - prow_z
            d2 = dx * dx + dy * dy + dz * dz
            close = (d2 < _CUT2).astype(f32)
            extc = jnp.sum(close * keep_row[...], axis=1, keepdims=True)
            ext01 = (extc > 0.0).astype(f32)
            ext_row = lax.dot_general(
                ext01, ident, (((0,), (0,)), ((), ())),
                preferred_element_type=f32)
            jrx = prow_x[0:1, s0:s0 + _BLK]
            jry = prow_y[0:1, s0:s0 + _BLK]
            jrz = prow_z[0:1, s0:s0 + _BLK]
            bx = icx - jrx
            by = icy - jry
            bz = icz - jrz
            bb = bx * bx + by * by + bz * bz
            trilc = ((bb < _CUT2) & (lane_b < sub_b)).astype(f32)
            kb0 = prow_m[0:1, s0:s0 + _BLK] * (1.0 - ext_row)

            def w_cond(st):
                return st[1]

            def w_body(st):
                kb, _ = st
                sup = lax.dot_general(kb, trilc, (((1,), (1,)), ((), ())),
                                      preferred_element_type=f32)
                kbn = kb0 * (sup == 0.0).astype(f32)
                chg = jnp.sum(jnp.abs(kbn - kb)) > 0.0
                return kbn, chg

            kb, _ = lax.while_loop(w_cond, w_body, (kb0, True))
            keep_row[0:1, s0:s0 + _BLK] = kb

    keep = keep_row[...]

    def t_tile(tt, carry):
        curmin, curarg = carry
        s = pl.ds(tt * _BLK, _BLK)
        ct = tg_cols[0, s, :]
        tcx = _chan_col(ct, 0)
        tcy = _chan_col(ct, 1)
        tcz = _chan_col(ct, 2)
        tcm = _chan_col(ct, 3)
        dx = tcx - prow_x
        dy = tcy - prow_y
        dz = tcz - prow_z
        d2 = dx * dx + dy * dy + dz * dz
        valid = tcm * keep
        d2m = jnp.where(valid > 0.0, d2, _BIG)
        tmin = jnp.min(d2m, axis=0, keepdims=True)
        tloc = jnp.min(jnp.where(d2m == tmin, sub_b.astype(f32), 1e9),
                       axis=0, keepdims=True)
        tglob = tloc + jnp.float32(1.0) * (tt * _BLK)
        upd = tmin < curmin
        curarg = jnp.where(upd, tglob, curarg)
        curmin = jnp.where(upd, tmin, curmin)
        rmin = jnp.min(d2m, axis=1, keepdims=True)
        ploc = jnp.min(jnp.where(d2m == rmin, lane_np.astype(f32), 1e9),
                       axis=1, keepdims=True)
        tgbest[s, :] = ploc
        tgd2[s, :] = rmin
        return curmin, curarg

    init = (jnp.full((1, _NP), _BIG, f32), jnp.zeros((1, _NP), f32))
    _, pdbest = lax.fori_loop(0, _NB, t_tile, init)

    pd_mat = pd_cols[0]

    def t3(tt, carry):
        angsum, wcnt, m0, m1, c0, c1 = carry
        s = pl.ds(tt * _BLK, _BLK)
        tgb = tgbest[s, :]
        td2 = tgd2[s, :]
        ct = tg_cols[0, s, :]
        tcz = _chan_col(ct, 2)
        tcm = _chan_col(ct, 3)
        sel = (lane_np.astype(f32) == tgb).astype(f32)
        tgi = sub_b.astype(f32) + jnp.float32(1.0) * (tt * _BLK)
        mut = jnp.sum(sel * (pdbest == tgi).astype(f32), axis=1, keepdims=True)
        w = ((mut > 0.0) & (td2 < _MATCH2)).astype(f32)

        gath = lax.dot_general(sel, pd_mat, (((1,), (0,)), ((), ())),
                               preferred_element_type=f32)
        angs = jnp.zeros((_BLK, 1), f32)
        for i in range(2):
            tr = [_chan_col(ct, 4 + 3 * i + c) for c in range(3)]
            gr = [_chan_col(gath, 4 + 3 * i + c) for c in range(3)]
            a_i = gr[0] * tr[0] + gr[1] * tr[1] + gr[2] * tr[2]
            n_i = _chan_col(gath, 10 + i)
            tn_i = jnp.sqrt(tr[0] * tr[0] + tr[1] * tr[1] + tr[2] * tr[2])
            cos = a_i / (n_i * tn_i)
            ang = _acos(jnp.clip(cos, -1.0, 1.0)) / _DEG
            ang = jnp.where(ang != ang, jnp.float32(90.0), ang)
            angs = angs + ang
        angsum = angsum + jnp.sum(w * angs)
        wcnt = wcnt + jnp.sum(w)
        zin0 = ((tcz >= _SPLIT[0]) & (tcz < _SPLIT[1])).astype(f32)
        zin1 = ((tcz >= _SPLIT[1]) & (tcz < _SPLIT[2])).astype(f32)
        m0 = m0 + jnp.sum(w * zin0)
        m1 = m1 + jnp.sum(w * zin1)
        c0 = c0 + jnp.sum(tcm * zin0)
        c1 = c1 + jnp.sum(tcm * zin1)
        return angsum, wcnt, m0, m1, c0, c1

    z6 = (jnp.float32(0.0),) * 6
    angsum, wcnt, m0, m1, tgc0, tgc1 = lax.fori_loop(0, _NB, t3, z6)

    pz0 = ((prow_z >= _SPLIT[0]) & (prow_z < _SPLIT[1])).astype(f32)
    pz1 = ((prow_z >= _SPLIT[1]) & (prow_z < _SPLIT[2])).astype(f32)
    pdc0 = jnp.sum(keep * pz0)
    pdc1 = jnp.sum(keep * pz1)

    wsum = wcnt * 2.0
    ang_mean = jnp.where(wsum > 0.0, angsum / jnp.maximum(wsum, 1.0),
                         jnp.float32(90.0))

    r = lax.broadcasted_iota(jnp.int32, (8, 128), 0)
    c = lax.broadcasted_iota(jnp.int32, (8, 128), 1)
    vals = ((0, 0, m0), (0, 1, pdc0 - m0), (0, 2, tgc0 - m0), (0, 3, ang_mean),
            (1, 0, m1), (1, 1, pdc1 - m1), (1, 2, tgc1 - m1), (1, 3, ang_mean))
    res = jnp.zeros((8, 128), f32)
    for (ri, ci, v) in vals:
        res = jnp.where((r == ri) & (c == ci), v, res)
    out[0] = res


@functools.partial(jax.jit, static_argnames=())
def kernel(preds, targs):
    B = preds.shape[0]
    f32 = jnp.float32
    pred = preds.reshape(B, _N, 10)
    targ = targs.reshape(B, _N, 10)

    gx, gy, gz = jnp.meshgrid(jnp.arange(25), jnp.arange(25), jnp.arange(8),
                              indexing='ij')
    grid = jnp.stack([gx, gy, gz], axis=-1).reshape(-1, 3).astype(f32)
    cell = jnp.asarray([1.0, 1.0, 0.5], dtype=f32)

    def fields(box):
        conf = box[..., 0]
        off = jax.nn.sigmoid(box[..., 1:4])
        pos = (grid + off) * cell
        R = box[..., 4:10]
        return conf, pos, R

    conf_p, pos_p, R_p = fields(pred)
    conf_t, pos_t, R_t = fields(targ)
    mask_p = (conf_p > 0.0).astype(f32)
    mask_t = (conf_t > 0.5).astype(f32)

    order = jnp.argsort(-conf_p, axis=1)
    pos_p = jnp.take_along_axis(pos_p, order[..., None], axis=1)
    R_p = jnp.take_along_axis(R_p, order[..., None], axis=1)
    mask_p = jnp.take_along_axis(mask_p, order, axis=1)
    nrm_p = jnp.linalg.norm(R_p.reshape(B, _N, 2, 3), axis=-1)

    pad = ((0, 0), (0, _NP - _N), (0, 0))

    pd_chan = jnp.concatenate(
        [pos_p, mask_p[..., None], R_p, nrm_p,
         jnp.zeros((B, _N, 4), f32)], axis=-1)
    pd_chan = jnp.pad(pd_chan, pad)
    pd_rows = jnp.swapaxes(pd_chan, 1, 2)
    pd_cols = pd_chan

    tg_chan = jnp.concatenate(
        [pos_t, mask_t[..., None], R_t,
         jnp.zeros((B, _N, 6), f32)], axis=-1)
    tg_cols = jnp.pad(tg_chan, pad)

    out = pl.pallas_call(
        _analyse_kernel,
        grid=(B,),
        in_specs=[
            pl.BlockSpec((1, 16, _NP), lambda b: (b, 0, 0)),
            pl.BlockSpec((1, _NP, 16), lambda b: (b, 0, 0)),
            pl.BlockSpec((1, _NP, 16), lambda b: (b, 0, 0)),
        ],
        out_specs=pl.BlockSpec((1, 8, 128), lambda b: (b, 0, 0)),
        out_shape=jax.ShapeDtypeStruct((B, 8, 128), f32),
        scratch_shapes=[
            pltpu.VMEM((1, _NP), f32),
            pltpu.VMEM((_NP, 1), f32),
            pltpu.VMEM((_NP, 1), f32),
        ],
    )(pd_rows, pd_cols, tg_cols)

    return out[:, None, :2, :4]
